# baseline (speedup 1.0000x reference)
import functools

import jax
import jax.numpy as jnp
import numpy as np
from jax import lax
from jax.experimental import pallas as pl
from jax.experimental.pallas import tpu as pltpu
from jax.experimental.pallas import tpu_sc as plsc

N_E = 1024
E_DIM = 256
N_CHANNEL = 4
D_MODEL = 1024
BETA = 0.25

_TOK_BLK = 512
_ROW_BLK = _TOK_BLK * N_CHANNEL

_TINY = np.float32(1.1754944e-38)


@functools.lru_cache(maxsize=1)
def _uniform_table(n_rows: int, n_cols: int):
    rot1 = (13, 15, 26, 6)
    rot2 = (17, 29, 16, 24)
    ks = (np.uint32(0), np.uint32(42),
          np.uint32(0) ^ np.uint32(42) ^ np.uint32(0x1BD11BDA))
    with np.errstate(over="ignore"):
        x1 = np.arange(n_rows * n_cols, dtype=np.uint32) + ks[1]
        x0 = np.full_like(x1, ks[0])
        for i in range(5):
            for r in (rot1 if i % 2 == 0 else rot2):
                x0 = x0 + x1
                x1 = (x1 << np.uint32(r)) | (x1 >> np.uint32(32 - r))
                x1 ^= x0
            x0 = x0 + ks[(i + 1) % 3]
            x1 = x1 + (ks[(i + 2) % 3] + np.uint32(i + 1))
    bits = x0 ^ x1
    f = ((bits >> np.uint32(9)) | np.uint32(0x3F800000)).view(np.float32)
    return (f - np.float32(1.0)).reshape(n_rows, n_cols)


def _sample_kernel(u_ref, z_ref, fc0_w_ref, fc0_b_ref, emb_ref, emb_m2_ref,
                   idx_ref, loss_ref):
    i = pl.program_id(0)

    emb = emb_ref[...]
    e_blk = jax.lax.dot_general(
        z_ref[...], fc0_w_ref[...], (((1,), (1,)), ((), ())),
        preferred_element_type=jnp.float32) + fc0_b_ref[...]
    cz = e_blk.reshape(_ROW_BLK, E_DIM)

    s_z = jnp.sum(cz * cz, axis=1, keepdims=True)
    s_e = jnp.sum(emb * emb, axis=1)[None, :]
    cross_m2 = jax.lax.dot_general(
        cz, emb_m2_ref[...], (((1,), (1,)), ((), ())),
        preferred_element_type=jnp.float32)
    logits = (s_z + s_e) + cross_m2
    ls = logits - jnp.max(logits, axis=1, keepdims=True)

    g = -jnp.log(-jnp.log(jnp.maximum(u_ref[...], _TINY)))
    y = g + ls
    m = jnp.max(y, axis=1, keepdims=True)
    colf = jax.lax.broadcasted_iota(jnp.int32, (_ROW_BLK, N_E), 1)
    idx = jnp.min(jnp.where(y == m, colf, jnp.int32(N_E)), axis=1)
    sel = colf == idx[:, None]
    idx_ref[...] = idx.reshape(1, 1, _ROW_BLK)

    part = jnp.sum(jnp.where(sel, logits, jnp.float32(0.0)))

    @pl.when(i == 0)
    def _():
        loss_ref[...] = jnp.zeros((1, 1), jnp.float32)

    loss_ref[...] += part.reshape(1, 1)


def _fc1_kernel(czq_ref, fc1_w_ref, fc1_b_ref, zq_ref):
    q_blk = czq_ref[...].reshape(_TOK_BLK, N_CHANNEL * E_DIM)
    zq_ref[...] = jax.lax.dot_general(
        q_blk, fc1_w_ref[...], (((1,), (1,)), ((), ())),
        preferred_element_type=jnp.float32) + fc1_b_ref[...]


def _sc_gather(n_rows):
    info = plsc.get_sparse_core_info()
    n_workers = info.num_cores * info.num_subcores
    rows_per_w = n_rows // n_workers
    chunk = 256
    n_chunks = rows_per_w // chunk
    mesh = plsc.VectorSubcoreMesh(core_axis_name="c", subcore_axis_name="s")

    @functools.partial(
        pl.kernel, mesh=mesh,
        out_type=jax.ShapeDtypeStruct((n_rows, E_DIM), jnp.float32),
        scratch_types=[
            pltpu.VMEM((chunk,), jnp.int32),
            pltpu.VMEM((chunk, E_DIM), jnp.float32),
            pltpu.SemaphoreType.DMA,
        ],
    )
    def gather(table_hbm, idx_hbm, out_hbm, idx_v, rows_v, sem):
        wid = lax.axis_index("s") * info.num_cores + lax.axis_index("c")
        base = wid * rows_per_w
        for c in range(n_chunks):
            off = base + c * chunk
            pltpu.sync_copy(idx_hbm.at[pl.ds(off, chunk)], idx_v)
            pltpu.async_copy(table_hbm.at[idx_v], rows_v, sem).wait()
            pltpu.sync_copy(rows_v, out_hbm.at[pl.ds(off, chunk)])

    return gather


@functools.partial(jax.jit, static_argnums=())
def kernel(z, fc0_w, fc0_b, fc1_w, fc1_b, emb):
    n_batch, n_seq, d_model = z.shape
    n_tok = n_batch * n_seq
    n_rows = n_tok * N_CHANNEL
    z2 = z.reshape(n_tok, d_model)
    n_blocks = n_tok // _TOK_BLK

    u = jnp.asarray(_uniform_table(n_rows, N_E))

    idx3, loss_sum = pl.pallas_call(
        _sample_kernel,
        grid=(n_blocks,),
        in_specs=[
            pl.BlockSpec((_ROW_BLK, N_E), lambda i: (i, 0)),
            pl.BlockSpec((_TOK_BLK, d_model), lambda i: (i, 0)),
            pl.BlockSpec((D_MODEL, D_MODEL), lambda i: (0, 0)),
            pl.BlockSpec((1, D_MODEL), lambda i: (0, 0)),
            pl.BlockSpec((N_E, E_DIM), lambda i: (0, 0)),
            pl.BlockSpec((N_E, E_DIM), lambda i: (0, 0)),
        ],
        out_specs=[
            pl.BlockSpec((1, 1, _ROW_BLK), lambda i: (i, 0, 0)),
            pl.BlockSpec((1, 1), lambda i: (0, 0)),
        ],
        out_shape=[
            jax.ShapeDtypeStruct((n_blocks, 1, _ROW_BLK), jnp.int32),
            jax.ShapeDtypeStruct((1, 1), jnp.float32),
        ],
        compiler_params=pltpu.CompilerParams(
            dimension_semantics=("arbitrary",),
        ),
    )(u, z2, fc0_w, fc0_b.reshape(1, -1), emb, jnp.float32(-2.0) * emb)

    idx = idx3.reshape(n_rows)
    czq = _sc_gather(n_rows)(emb, idx)

    zq = pl.pallas_call(
        _fc1_kernel,
        grid=(n_blocks,),
        in_specs=[
            pl.BlockSpec((_ROW_BLK, E_DIM), lambda i: (i, 0)),
            pl.BlockSpec((D_MODEL, D_MODEL), lambda i: (0, 0)),
            pl.BlockSpec((1, D_MODEL), lambda i: (0, 0)),
        ],
        out_specs=pl.BlockSpec((_TOK_BLK, d_model), lambda i: (i, 0)),
        out_shape=jax.ShapeDtypeStruct((n_tok, d_model), jnp.float32),
    )(czq, fc1_w, fc1_b.reshape(1, -1))

    mean = loss_sum[0, 0] / jnp.float32(n_rows * E_DIM)
    loss = mean + jnp.float32(BETA) * mean
    return (loss, zq.reshape(n_batch, n_seq, d_model))
